# baseline (speedup 1.0000x reference)
import functools

import jax
import jax.numpy as jnp
from jax import lax
from jax.experimental import pallas as pl
from jax.experimental.pallas import tpu as pltpu
from jax.experimental.pallas import tpu_sc as plsc

_B, _T, _H = 1, 2048, 1024
_M, _D = 4096, 256
_G = 16
_NH = 4
_TOP_K = 32

_TT = 128
_R = _TT * _NH
_CH = 1024
_NCH = _M // _CH

_bf = jnp.bfloat16
_f32 = jnp.float32


def _dot(a, b):
    return jax.lax.dot_general(
        a.astype(_bf), b if b.dtype == _bf else b.astype(_bf),
        (((1,), (0,)), ((), ())), preferred_element_type=_f32)


def _main_body(hid_ref, bel_ref, bias_ref, nw_ref, wcat_ref, bg_ref,
               ho_ref, wv_ref, conf_ref, util_ref, attn_ref, retr_ref,
               obs_ref, mass_ref, scr_ref):
    t = pl.program_id(0)

    @pl.when(t == 0)
    def _init():
        mass_ref[...] = jnp.zeros_like(mass_ref)

    x = hid_ref[0]
    v = jnp.mean(x * x, axis=-1, keepdims=True)
    normed = x * jax.lax.rsqrt(v + 1e-6) * nw_ref[...]

    q = _dot(normed, wcat_ref[:, 0:_H])
    q4 = (q * 0.0625).reshape(_R, _D).astype(_bf)

    mcs, sums = [], []
    for c in range(_NCH):
        sl = slice(c * _CH, (c + 1) * _CH)
        raw = jax.lax.dot_general(
            q4, bel_ref[sl, :],
            (((1,), (1,)), ((), ())), preferred_element_type=_f32)
        s = raw + bias_ref[:, sl]
        m_c = jnp.max(s, axis=-1, keepdims=True)
        e = jnp.exp(s - m_c)
        sums.append(jnp.sum(e, axis=-1, keepdims=True))
        mcs.append(m_c)
        scr_ref[:, sl] = e
    m = mcs[0]
    for c in range(1, _NCH):
        m = jnp.maximum(m, mcs[c])
    corr = [jnp.exp(mc - m) for mc in mcs]
    total = corr[0] * sums[0]
    for c in range(1, _NCH):
        total = total + corr[c] * sums[c]
    sinv = 1.0 / total
    racc = jnp.zeros((_R, _D), _f32)
    for c in range(_NCH):
        sl = slice(c * _CH, (c + 1) * _CH)
        a = scr_ref[:, sl] * (corr[c] * sinv)
        attn_ref[0, :, :, sl] = a.reshape(_TT, _NH, _CH)
        mass_ref[:, sl] += jnp.sum(a, axis=0, keepdims=True)
        racc = racc + _dot(a, bel_ref[sl, :])
    retr_ref[0] = racc.reshape(_TT, _NH, _D)

    rflat = racc.reshape(_TT, _NH * _D)
    binfo = _dot(rflat, wcat_ref[:, _H:2 * _H])
    gc = _dot(normed, wcat_ref[:, 3 * _H + 2 * _D:3 * _H + 2 * _D + 2])
    gate = jax.nn.sigmoid(gc[:, 0:1] + bg_ref[0, 0])
    conf_ref[0] = jax.nn.sigmoid(gc[:, 1:2])
    ho_ref[0] = x + binfo * gate
    util_ref[0] = _dot(normed, wcat_ref[:, 2 * _H:3 * _H])
    obs_ref[0] = _dot(normed, wcat_ref[:, 3 * _H:3 * _H + _D])
    wv_ref[0] = _dot(normed, wcat_ref[:, 3 * _H + _D:3 * _H + 2 * _D])


_NSUB = 16
_SEG = _M // _NSUB
_NV = _SEG // 16
_NEG = -3.0e38
_BIG = 2 ** 30


def _vmaxall(x):
    s = plsc.cummax(x)
    return plsc.cummax(lax.rev(s, (0,)))


def _vminall_i32(x):
    return -_vmaxall(-x)


def _sc_topk(mass_hbm, out_hbm, stagev_hbm, stagei_hbm,
             xv, bufv, bufi, mv, mi, outb):
    cid = lax.axis_index("c")
    sid = lax.axis_index("s")
    iota = lax.iota(jnp.int32, 16)

    @pl.when(cid == 0)
    def _():
        base = sid * _SEG
        pltpu.sync_copy(mass_hbm.at[pl.ds(base, _SEG)], xv)

        def local_step(k, res):
            rv0, rv1, ri0, ri1 = res
            vm = xv[pl.ds(0, 16)]
            for v in range(1, _NV):
                vm = jnp.maximum(vm, xv[pl.ds(v * 16, 16)])
            m = _vmaxall(vm)
            cand = jnp.full((16,), _BIG, jnp.int32)
            for v in range(_NV):
                xvv = xv[pl.ds(v * 16, 16)]
                cand = jnp.minimum(
                    cand, jnp.where(xvv == m, iota + (base + v * 16), _BIG))
            gi = _vminall_i32(cand)
            for v in range(_NV):
                xvv = xv[pl.ds(v * 16, 16)]
                xv[pl.ds(v * 16, 16)] = jnp.where(
                    iota + (base + v * 16) == gi, _NEG, xvv)
            lo = (iota == (k % 16)) & (k < 16)
            hi = (iota == (k % 16)) & (k >= 16)
            rv0 = jnp.where(lo, m, rv0)
            rv1 = jnp.where(hi, m, rv1)
            ri0 = jnp.where(lo, gi, ri0)
            ri1 = jnp.where(hi, gi, ri1)
            return (rv0, rv1, ri0, ri1)

        z16f = jnp.full((16,), _NEG, _f32)
        z16i = jnp.zeros((16,), jnp.int32)
        rv0, rv1, ri0, ri1 = lax.fori_loop(
            0, _TOP_K, local_step, (z16f, z16f, z16i, z16i))
        bufv[pl.ds(0, 16)] = rv0
        bufv[pl.ds(16, 16)] = rv1
        bufi[pl.ds(0, 16)] = ri0
        bufi[pl.ds(16, 16)] = ri1
        pltpu.sync_copy(bufv, stagev_hbm.at[pl.ds(sid * _TOP_K, _TOP_K)])
        pltpu.sync_copy(bufi, stagei_hbm.at[pl.ds(sid * _TOP_K, _TOP_K)])
        plsc.subcore_barrier()

        @pl.when(sid == 0)
        def _merge():
            pltpu.sync_copy(stagev_hbm, mv)
            pltpu.sync_copy(stagei_hbm, mi)
            npool = _NSUB * _TOP_K // 16

            def merge_step(k, res):
                rv0, rv1, ri0, ri1 = res
                vm = mv[pl.ds(0, 16)]
                for v in range(1, npool):
                    vm = jnp.maximum(vm, mv[pl.ds(v * 16, 16)])
                m = _vmaxall(vm)
                cand = jnp.full((16,), _BIG, jnp.int32)
                for v in range(npool):
                    vv = mv[pl.ds(v * 16, 16)]
                    iv = mi[pl.ds(v * 16, 16)]
                    cand = jnp.minimum(cand, jnp.where(vv == m, iv, _BIG))
                gi = _vminall_i32(cand)
                for v in range(npool):
                    vv = mv[pl.ds(v * 16, 16)]
                    iv = mi[pl.ds(v * 16, 16)]
                    mv[pl.ds(v * 16, 16)] = jnp.where(
                        (vv == m) & (iv == gi), _NEG, vv)
                lo = (iota == (k % 16)) & (k < 16)
                hi = (iota == (k % 16)) & (k >= 16)
                rv0 = jnp.where(lo, m, rv0)
                rv1 = jnp.where(hi, m, rv1)
                ri0 = jnp.where(lo, gi, ri0)
                ri1 = jnp.where(hi, gi, ri1)
                return (rv0, rv1, ri0, ri1)

            _, _, ri0, ri1 = lax.fori_loop(
                0, _TOP_K, merge_step, (z16f, z16f, z16i, z16i))
            outb[pl.ds(0, 16)] = ri0
            outb[pl.ds(16, 16)] = ri1
            pltpu.sync_copy(outb, out_hbm)


def _topk_body(mass_ref, idx_ref):
    m = mass_ref[...]
    iota = jax.lax.broadcasted_iota(jnp.int32, (1, _M), 1)
    lanes = jax.lax.broadcasted_iota(jnp.int32, (1, _TOP_K), 1)

    def step(i, carry):
        m, inds = carry
        cm = jnp.max(m)
        idx = jnp.min(jnp.where(m == cm, iota, jnp.int32(2 ** 30)))
        inds = jnp.where(lanes == i, idx, inds)
        m = jnp.where(iota == idx, -jnp.inf, m)
        return (m, inds)

    _, inds = jax.lax.fori_loop(
        0, _TOP_K, step, (m, jnp.zeros((1, _TOP_K), jnp.int32)))
    idx_ref[...] = inds


def kernel(hidden, beliefs, goal_embeddings, goal_priorities, norm_weight,
           depth_bias, W_q, W_out, W_gate, b_gate, W_util, W_obs, W_write,
           W_conf):
    B, T, H, M, D, G, NH = _B, _T, _H, _M, _D, _G, _NH
    goal_bias = (beliefs @ goal_embeddings.T) @ goal_priorities / G
    bias_row = (depth_bias[0] + goal_bias).reshape(1, M)

    bel_bf = beliefs.astype(_bf)
    wcat_bf = jnp.concatenate(
        [W_q, W_out, W_util, W_obs, W_write, W_gate, W_conf],
        axis=1).astype(_bf)
    nw = norm_weight.reshape(1, H)
    bg = b_gate.reshape(1, 1)

    grid = T // _TT

    out_shapes = (
        jax.ShapeDtypeStruct((B, T, H), _f32),
        jax.ShapeDtypeStruct((B, T, D), _f32),
        jax.ShapeDtypeStruct((B, T, 1), _f32),
        jax.ShapeDtypeStruct((B, T, H), _f32),
        jax.ShapeDtypeStruct((B, T, NH, M), _f32),
        jax.ShapeDtypeStruct((B, T, NH, D), _f32),
        jax.ShapeDtypeStruct((B, T, D), _f32),
        jax.ShapeDtypeStruct((1, M), _f32),
    )
    full = lambda shape: pl.BlockSpec(shape, lambda t: (0,) * len(shape))
    row = lambda last: pl.BlockSpec((1, _TT, last), lambda t: (0, t, 0))
    row4 = lambda last: pl.BlockSpec((1, _TT, _NH, last),
                                     lambda t: (0, t, 0, 0))

    outs = pl.pallas_call(
        _main_body,
        grid=(grid,),
        in_specs=[
            row(H),
            full((M, D)),
            full((1, M)),
            full((1, H)),
            full((H, 3 * H + 2 * D + 2)),
            full((1, 1)),
        ],
        out_specs=[
            row(H), row(D), row(1), row(H),
            row4(M), row4(D), row(D),
            full((1, M)),
        ],
        out_shape=out_shapes,
        scratch_shapes=[pltpu.VMEM((_R, _M), _f32)],
    )(hidden, bel_bf, bias_row, nw, wcat_bf, bg)

    (hidden_out, write_vec, confidence, utility_logits, attn_weights,
     retrieved, obs_vectors, mass) = outs

    topk_call = pl.kernel(
        _sc_topk,
        out_type=(
            jax.ShapeDtypeStruct((_TOP_K,), jnp.int32),
            jax.ShapeDtypeStruct((_NSUB * _TOP_K,), _f32),
            jax.ShapeDtypeStruct((_NSUB * _TOP_K,), jnp.int32),
        ),
        scratch_types=[
            pltpu.VMEM((_SEG,), _f32),
            pltpu.VMEM((_TOP_K,), _f32),
            pltpu.VMEM((_TOP_K,), jnp.int32),
            pltpu.VMEM((_NSUB * _TOP_K,), _f32),
            pltpu.VMEM((_NSUB * _TOP_K,), jnp.int32),
            pltpu.VMEM((_TOP_K,), jnp.int32),
        ],
        mesh=plsc.VectorSubcoreMesh(core_axis_name="c", subcore_axis_name="s"),
        compiler_params=pltpu.CompilerParams(needs_layout_passes=False),
    )
    read_indices, _, _ = topk_call(mass.reshape(M))

    return (hidden_out, write_vec, confidence, utility_logits, read_indices,
            attn_weights, retrieved, obs_vectors)


# speedup vs baseline: 1.0569x; 1.0569x over previous
import functools

import jax
import jax.numpy as jnp
from jax import lax
from jax.experimental import pallas as pl
from jax.experimental.pallas import tpu as pltpu
from jax.experimental.pallas import tpu_sc as plsc

_B, _T, _H = 1, 2048, 1024
_M, _D = 4096, 256
_G = 16
_NH = 4
_TOP_K = 32

_TT = 128
_R = _TT * _NH
_CH = 1024
_NCH = _M // _CH

_bf = jnp.bfloat16
_f32 = jnp.float32


def _dot(a, b):
    return jax.lax.dot_general(
        a.astype(_bf), b if b.dtype == _bf else b.astype(_bf),
        (((1,), (0,)), ((), ())), preferred_element_type=_f32)


def _main_body(hid_ref, belT_ref, bel_ref, bias_ref, nw_ref, wcat_ref,
               bg_ref, ho_ref, wv_ref, conf_ref, util_ref, attn_ref,
               retr_ref, obs_ref, mass_ref, scr_ref):
    t = pl.program_id(0)

    @pl.when(t == 0)
    def _init():
        mass_ref[...] = jnp.zeros_like(mass_ref)

    x = hid_ref[0]
    v = jnp.mean(x * x, axis=-1, keepdims=True)
    normed = x * jax.lax.rsqrt(v + 1e-6) * nw_ref[...]

    q = _dot(normed, wcat_ref[:, 0:_H])
    q4 = (q * 0.0625).reshape(_R, _D).astype(_bf)

    mcs, sums = [], []
    for c in range(_NCH):
        sl = slice(c * _CH, (c + 1) * _CH)
        raw = jax.lax.dot_general(
            q4, belT_ref[:, sl],
            (((1,), (0,)), ((), ())), preferred_element_type=_f32)
        s = raw + bias_ref[:, sl]
        m_c = jnp.max(s, axis=-1, keepdims=True)
        e = jnp.exp(s - m_c)
        sums.append(jnp.sum(e, axis=-1, keepdims=True))
        mcs.append(m_c)
        scr_ref[:, sl] = e
    m = mcs[0]
    for c in range(1, _NCH):
        m = jnp.maximum(m, mcs[c])
    corr = [jnp.exp(mc - m) for mc in mcs]
    total = corr[0] * sums[0]
    for c in range(1, _NCH):
        total = total + corr[c] * sums[c]
    sinv = 1.0 / total
    racc = jnp.zeros((_R, _D), _f32)
    for c in range(_NCH):
        sl = slice(c * _CH, (c + 1) * _CH)
        a = scr_ref[:, sl] * (corr[c] * sinv)
        attn_ref[0, :, :, sl] = a.reshape(_TT, _NH, _CH)
        mass_ref[:, sl] += jnp.sum(a, axis=0, keepdims=True)
        racc = racc + _dot(a, bel_ref[sl, :])
    retr_ref[0] = racc.reshape(_TT, _NH, _D)

    rflat = racc.reshape(_TT, _NH * _D)
    binfo = _dot(rflat, wcat_ref[:, _H:2 * _H])
    gc = _dot(normed, wcat_ref[:, 3 * _H + 2 * _D:3 * _H + 2 * _D + 2])
    gate = jax.nn.sigmoid(gc[:, 0:1] + bg_ref[0, 0])
    conf_ref[0] = jax.nn.sigmoid(gc[:, 1:2])
    ho_ref[0] = x + binfo * gate
    util_ref[0] = _dot(normed, wcat_ref[:, 2 * _H:3 * _H])
    obs_ref[0] = _dot(normed, wcat_ref[:, 3 * _H:3 * _H + _D])
    wv_ref[0] = _dot(normed, wcat_ref[:, 3 * _H + _D:3 * _H + 2 * _D])


_NSUB = 16
_SEG = _M // _NSUB
_NV = _SEG // 16
_NEG = -3.0e38
_BIG = 2 ** 30


def _prep_body(wq, wout, wutil, wobs, wwrite, wgate, wconf, bel,
               wcat_o, bel_o, belT_o):
    wcat_o[:, 0:_H] = wq[...].astype(_bf)
    wcat_o[:, _H:2 * _H] = wout[...].astype(_bf)
    wcat_o[:, 2 * _H:3 * _H] = wutil[...].astype(_bf)
    wcat_o[:, 3 * _H:3 * _H + _D] = wobs[...].astype(_bf)
    wcat_o[:, 3 * _H + _D:3 * _H + 2 * _D] = wwrite[...].astype(_bf)
    wcat_o[:, 3 * _H + 2 * _D:3 * _H + 2 * _D + 1] = wgate[...].astype(_bf)
    wcat_o[:, 3 * _H + 2 * _D + 1:3 * _H + 2 * _D + 2] = wconf[...].astype(_bf)
    b = bel[...].astype(_bf)
    bel_o[...] = b
    belT_o[...] = b.T


def _vmaxall(x):
    s = plsc.cummax(x)
    return plsc.cummax(lax.rev(s, (0,)))


def _vminall_i32(x):
    return -_vmaxall(-x)


def _sc_topk(mass_hbm, out_hbm, stagev_hbm, stagei_hbm,
             xv, bufv, bufi, mv, mi, outb):
    cid = lax.axis_index("c")
    sid = lax.axis_index("s")
    iota = lax.iota(jnp.int32, 16)

    @pl.when(cid == 0)
    def _():
        base = sid * _SEG
        pltpu.sync_copy(mass_hbm.at[pl.ds(base, _SEG)], xv)

        def local_step(k, res):
            rv0, rv1, ri0, ri1 = res
            vm = xv[pl.ds(0, 16)]
            for v in range(1, _NV):
                vm = jnp.maximum(vm, xv[pl.ds(v * 16, 16)])
            m = _vmaxall(vm)
            cand = jnp.full((16,), _BIG, jnp.int32)
            for v in range(_NV):
                xvv = xv[pl.ds(v * 16, 16)]
                cand = jnp.minimum(
                    cand, jnp.where(xvv == m, iota + (base + v * 16), _BIG))
            gi = _vminall_i32(cand)
            for v in range(_NV):
                xvv = xv[pl.ds(v * 16, 16)]
                xv[pl.ds(v * 16, 16)] = jnp.where(
                    iota + (base + v * 16) == gi, _NEG, xvv)
            lo = (iota == (k % 16)) & (k < 16)
            hi = (iota == (k % 16)) & (k >= 16)
            rv0 = jnp.where(lo, m, rv0)
            rv1 = jnp.where(hi, m, rv1)
            ri0 = jnp.where(lo, gi, ri0)
            ri1 = jnp.where(hi, gi, ri1)
            return (rv0, rv1, ri0, ri1)

        z16f = jnp.full((16,), _NEG, _f32)
        z16i = jnp.zeros((16,), jnp.int32)
        rv0, rv1, ri0, ri1 = lax.fori_loop(
            0, _TOP_K, local_step, (z16f, z16f, z16i, z16i))
        bufv[pl.ds(0, 16)] = rv0
        bufv[pl.ds(16, 16)] = rv1
        bufi[pl.ds(0, 16)] = ri0
        bufi[pl.ds(16, 16)] = ri1
        pltpu.sync_copy(bufv, stagev_hbm.at[pl.ds(sid * _TOP_K, _TOP_K)])
        pltpu.sync_copy(bufi, stagei_hbm.at[pl.ds(sid * _TOP_K, _TOP_K)])
        plsc.subcore_barrier()

        @pl.when(sid == 0)
        def _merge():
            pltpu.sync_copy(stagev_hbm, mv)
            pltpu.sync_copy(stagei_hbm, mi)
            npool = _NSUB * _TOP_K // 16

            def merge_step(k, res):
                rv0, rv1, ri0, ri1 = res
                vm = mv[pl.ds(0, 16)]
                for v in range(1, npool):
                    vm = jnp.maximum(vm, mv[pl.ds(v * 16, 16)])
                m = _vmaxall(vm)
                cand = jnp.full((16,), _BIG, jnp.int32)
                for v in range(npool):
                    vv = mv[pl.ds(v * 16, 16)]
                    iv = mi[pl.ds(v * 16, 16)]
                    cand = jnp.minimum(cand, jnp.where(vv == m, iv, _BIG))
                gi = _vminall_i32(cand)
                for v in range(npool):
                    vv = mv[pl.ds(v * 16, 16)]
                    iv = mi[pl.ds(v * 16, 16)]
                    mv[pl.ds(v * 16, 16)] = jnp.where(
                        (vv == m) & (iv == gi), _NEG, vv)
                lo = (iota == (k % 16)) & (k < 16)
                hi = (iota == (k % 16)) & (k >= 16)
                rv0 = jnp.where(lo, m, rv0)
                rv1 = jnp.where(hi, m, rv1)
                ri0 = jnp.where(lo, gi, ri0)
                ri1 = jnp.where(hi, gi, ri1)
                return (rv0, rv1, ri0, ri1)

            _, _, ri0, ri1 = lax.fori_loop(
                0, _TOP_K, merge_step, (z16f, z16f, z16i, z16i))
            outb[pl.ds(0, 16)] = ri0
            outb[pl.ds(16, 16)] = ri1
            pltpu.sync_copy(outb, out_hbm)


def _topk_body(mass_ref, idx_ref):
    m = mass_ref[...]
    iota = jax.lax.broadcasted_iota(jnp.int32, (1, _M), 1)
    lanes = jax.lax.broadcasted_iota(jnp.int32, (1, _TOP_K), 1)

    def step(i, carry):
        m, inds = carry
        cm = jnp.max(m)
        idx = jnp.min(jnp.where(m == cm, iota, jnp.int32(2 ** 30)))
        inds = jnp.where(lanes == i, idx, inds)
        m = jnp.where(iota == idx, -jnp.inf, m)
        return (m, inds)

    _, inds = jax.lax.fori_loop(
        0, _TOP_K, step, (m, jnp.zeros((1, _TOP_K), jnp.int32)))
    idx_ref[...] = inds


def kernel(hidden, beliefs, goal_embeddings, goal_priorities, norm_weight,
           depth_bias, W_q, W_out, W_gate, b_gate, W_util, W_obs, W_write,
           W_conf):
    B, T, H, M, D, G, NH = _B, _T, _H, _M, _D, _G, _NH
    goal_bias = (beliefs @ goal_embeddings.T) @ goal_priorities / G
    bias_row = (depth_bias[0] + goal_bias).reshape(1, M)

    wcat_bf, bel_bf, belT_bf = pl.pallas_call(
        _prep_body,
        out_shape=(
            jax.ShapeDtypeStruct((H, 3 * H + 2 * D + 2), _bf),
            jax.ShapeDtypeStruct((M, D), _bf),
            jax.ShapeDtypeStruct((D, M), _bf),
        ),
    )(W_q, W_out, W_util, W_obs, W_write, W_gate, W_conf, beliefs)
    nw = norm_weight.reshape(1, H)
    bg = b_gate.reshape(1, 1)

    grid = T // _TT

    out_shapes = (
        jax.ShapeDtypeStruct((B, T, H), _f32),
        jax.ShapeDtypeStruct((B, T, D), _f32),
        jax.ShapeDtypeStruct((B, T, 1), _f32),
        jax.ShapeDtypeStruct((B, T, H), _f32),
        jax.ShapeDtypeStruct((B, T, NH, M), _f32),
        jax.ShapeDtypeStruct((B, T, NH, D), _f32),
        jax.ShapeDtypeStruct((B, T, D), _f32),
        jax.ShapeDtypeStruct((1, M), _f32),
    )
    full = lambda shape: pl.BlockSpec(shape, lambda t: (0,) * len(shape))
    row = lambda last: pl.BlockSpec((1, _TT, last), lambda t: (0, t, 0))
    row4 = lambda last: pl.BlockSpec((1, _TT, _NH, last),
                                     lambda t: (0, t, 0, 0))

    outs = pl.pallas_call(
        _main_body,
        grid=(grid,),
        in_specs=[
            row(H),
            full((D, M)),
            full((M, D)),
            full((1, M)),
            full((1, H)),
            full((H, 3 * H + 2 * D + 2)),
            full((1, 1)),
        ],
        out_specs=[
            row(H), row(D), row(1), row(H),
            row4(M), row4(D), row(D),
            full((1, M)),
        ],
        out_shape=out_shapes,
        scratch_shapes=[pltpu.VMEM((_R, _M), _f32)],
    )(hidden, belT_bf, bel_bf, bias_row, nw, wcat_bf, bg)

    (hidden_out, write_vec, confidence, utility_logits, attn_weights,
     retrieved, obs_vectors, mass) = outs

    topk_call = pl.kernel(
        _sc_topk,
        out_type=(
            jax.ShapeDtypeStruct((_TOP_K,), jnp.int32),
            jax.ShapeDtypeStruct((_NSUB * _TOP_K,), _f32),
            jax.ShapeDtypeStruct((_NSUB * _TOP_K,), jnp.int32),
        ),
        scratch_types=[
            pltpu.VMEM((_SEG,), _f32),
            pltpu.VMEM((_TOP_K,), _f32),
            pltpu.VMEM((_TOP_K,), jnp.int32),
            pltpu.VMEM((_NSUB * _TOP_K,), _f32),
            pltpu.VMEM((_NSUB * _TOP_K,), jnp.int32),
            pltpu.VMEM((_TOP_K,), jnp.int32),
        ],
        mesh=plsc.VectorSubcoreMesh(core_axis_name="c", subcore_axis_name="s"),
        compiler_params=pltpu.CompilerParams(needs_layout_passes=False),
    )
    read_indices, _, _ = topk_call(mass.reshape(M))

    return (hidden_out, write_vec, confidence, utility_logits, read_indices,
            attn_weights, retrieved, obs_vectors)


# speedup vs baseline: 1.0758x; 1.0179x over previous
import functools

import jax
import jax.numpy as jnp
from jax import lax
from jax.experimental import pallas as pl
from jax.experimental.pallas import tpu as pltpu
from jax.experimental.pallas import tpu_sc as plsc

_B, _T, _H = 1, 2048, 1024
_M, _D = 4096, 256
_G = 16
_NH = 4
_TOP_K = 32

_TT = 128
_R = _TT * _NH
_CH = 1024
_NCH = _M // _CH

_bf = jnp.bfloat16
_f32 = jnp.float32


def _dot(a, b):
    return jax.lax.dot_general(
        a.astype(_bf), b if b.dtype == _bf else b.astype(_bf),
        (((1,), (0,)), ((), ())), preferred_element_type=_f32)


def _main_body(hid_ref, belT_ref, bel_ref, bias_ref, nw_ref, wcat_ref,
               bg_ref, ho_ref, wv_ref, conf_ref, util_ref, attn_ref,
               retr_ref, obs_ref, mass_ref, scr_ref):
    t = pl.program_id(0)

    @pl.when(t == 0)
    def _init():
        mass_ref[...] = jnp.zeros_like(mass_ref)

    x = hid_ref[0]
    v = jnp.mean(x * x, axis=-1, keepdims=True)
    normed = x * jax.lax.rsqrt(v + 1e-6) * nw_ref[...]

    q = _dot(normed, wcat_ref[:, 0:_H])
    q4 = (q * 0.0625).reshape(_R, _D).astype(_bf)

    mcs, sums = [], []
    for c in range(_NCH):
        sl = slice(c * _CH, (c + 1) * _CH)
        raw = jax.lax.dot_general(
            q4, belT_ref[:, sl],
            (((1,), (0,)), ((), ())), preferred_element_type=_f32)
        s = raw + bias_ref[:, sl]
        m_c = jnp.max(s, axis=-1, keepdims=True)
        e = jnp.exp(s - m_c)
        sums.append(jnp.sum(e, axis=-1, keepdims=True))
        mcs.append(m_c)
        scr_ref[:, sl] = e
    m = mcs[0]
    for c in range(1, _NCH):
        m = jnp.maximum(m, mcs[c])
    corr = [jnp.exp(mc - m) for mc in mcs]
    total = corr[0] * sums[0]
    for c in range(1, _NCH):
        total = total + corr[c] * sums[c]
    sinv = 1.0 / total
    racc = jnp.zeros((_R, _D), _f32)
    for c in range(_NCH):
        sl = slice(c * _CH, (c + 1) * _CH)
        a = scr_ref[:, sl] * (corr[c] * sinv)
        attn_ref[0, :, :, sl] = a.reshape(_TT, _NH, _CH)
        mass_ref[:, sl] += jnp.sum(a, axis=0, keepdims=True)
        racc = racc + _dot(a, bel_ref[sl, :])
    retr_ref[0] = racc.reshape(_TT, _NH, _D)

    rflat = racc.reshape(_TT, _NH * _D)
    binfo = _dot(rflat, wcat_ref[:, _H:2 * _H])
    gc = _dot(normed, wcat_ref[:, 3 * _H + 2 * _D:3 * _H + 2 * _D + 2])
    gate = jax.nn.sigmoid(gc[:, 0:1] + bg_ref[0, 0])
    conf_ref[0] = jax.nn.sigmoid(gc[:, 1:2])
    ho_ref[0] = x + binfo * gate
    util_ref[0] = _dot(normed, wcat_ref[:, 2 * _H:3 * _H])
    obs_ref[0] = _dot(normed, wcat_ref[:, 3 * _H:3 * _H + _D])
    wv_ref[0] = _dot(normed, wcat_ref[:, 3 * _H + _D:3 * _H + 2 * _D])


_NSUB = 16
_SEG = _M // _NSUB
_NV = _SEG // 16
_NEG = -3.0e38
_BIG = 2 ** 30


def _prep_body(wq, wout, wutil, wobs, wwrite, wgate, wconf, bel, ge, prio,
               db, wcat_o, bel_o, belT_o, bias_o):
    wcat_o[:, 0:_H] = wq[...].astype(_bf)
    wcat_o[:, _H:2 * _H] = wout[...].astype(_bf)
    wcat_o[:, 2 * _H:3 * _H] = wutil[...].astype(_bf)
    wcat_o[:, 3 * _H:3 * _H + _D] = wobs[...].astype(_bf)
    wcat_o[:, 3 * _H + _D:3 * _H + 2 * _D] = wwrite[...].astype(_bf)
    wcat_o[:, 3 * _H + 2 * _D:3 * _H + 2 * _D + 1] = wgate[...].astype(_bf)
    wcat_o[:, 3 * _H + 2 * _D + 1:3 * _H + 2 * _D + 2] = wconf[...].astype(_bf)
    b = bel[...].astype(_bf)
    bel_o[...] = b
    bT = b.T
    belT_o[...] = bT
    gb1 = jax.lax.dot_general(
        b, ge[...].astype(_bf),
        (((1,), (1,)), ((), ())), preferred_element_type=_f32)
    gb = jax.lax.dot_general(
        prio[...].astype(_bf), gb1.astype(_bf),
        (((1,), (1,)), ((), ())), preferred_element_type=_f32)
    bias_o[...] = gb * (1.0 / _G) + db[0, 0]


def _vmaxall(x):
    s = plsc.cummax(x)
    return plsc.cummax(lax.rev(s, (0,)))


def _vminall_i32(x):
    return -_vmaxall(-x)


def _sc_topk(mass_hbm, out_hbm, stagev_hbm, stagei_hbm,
             xv, bufv, bufi, mv, mi, outb):
    cid = lax.axis_index("c")
    sid = lax.axis_index("s")
    iota = lax.iota(jnp.int32, 16)

    @pl.when(cid == 0)
    def _():
        base = sid * _SEG
        pltpu.sync_copy(mass_hbm.at[pl.ds(base, _SEG)], xv)

        def local_step(k, res):
            rv0, rv1, ri0, ri1 = res
            vm = xv[pl.ds(0, 16)]
            for v in range(1, _NV):
                vm = jnp.maximum(vm, xv[pl.ds(v * 16, 16)])
            m = _vmaxall(vm)
            cand = jnp.full((16,), _BIG, jnp.int32)
            for v in range(_NV):
                xvv = xv[pl.ds(v * 16, 16)]
                cand = jnp.minimum(
                    cand, jnp.where(xvv == m, iota + (base + v * 16), _BIG))
            gi = _vminall_i32(cand)
            for v in range(_NV):
                xvv = xv[pl.ds(v * 16, 16)]
                xv[pl.ds(v * 16, 16)] = jnp.where(
                    iota + (base + v * 16) == gi, _NEG, xvv)
            lo = (iota == (k % 16)) & (k < 16)
            hi = (iota == (k % 16)) & (k >= 16)
            rv0 = jnp.where(lo, m, rv0)
            rv1 = jnp.where(hi, m, rv1)
            ri0 = jnp.where(lo, gi, ri0)
            ri1 = jnp.where(hi, gi, ri1)
            return (rv0, rv1, ri0, ri1)

        z16f = jnp.full((16,), _NEG, _f32)
        z16i = jnp.zeros((16,), jnp.int32)
        rv0, rv1, ri0, ri1 = lax.fori_loop(
            0, _TOP_K, local_step, (z16f, z16f, z16i, z16i))
        bufv[pl.ds(0, 16)] = rv0
        bufv[pl.ds(16, 16)] = rv1
        bufi[pl.ds(0, 16)] = ri0
        bufi[pl.ds(16, 16)] = ri1
        pltpu.sync_copy(bufv, stagev_hbm.at[pl.ds(sid * _TOP_K, _TOP_K)])
        pltpu.sync_copy(bufi, stagei_hbm.at[pl.ds(sid * _TOP_K, _TOP_K)])
        plsc.subcore_barrier()

        @pl.when(sid == 0)
        def _merge():
            pltpu.sync_copy(stagev_hbm, mv)
            pltpu.sync_copy(stagei_hbm, mi)
            npool = _NSUB * _TOP_K // 16

            def merge_step(k, res):
                rv0, rv1, ri0, ri1 = res
                vm = mv[pl.ds(0, 16)]
                for v in range(1, npool):
                    vm = jnp.maximum(vm, mv[pl.ds(v * 16, 16)])
                m = _vmaxall(vm)
                cand = jnp.full((16,), _BIG, jnp.int32)
                for v in range(npool):
                    vv = mv[pl.ds(v * 16, 16)]
                    iv = mi[pl.ds(v * 16, 16)]
                    cand = jnp.minimum(cand, jnp.where(vv == m, iv, _BIG))
                gi = _vminall_i32(cand)
                for v in range(npool):
                    vv = mv[pl.ds(v * 16, 16)]
                    iv = mi[pl.ds(v * 16, 16)]
                    mv[pl.ds(v * 16, 16)] = jnp.where(
                        (vv == m) & (iv == gi), _NEG, vv)
                lo = (iota == (k % 16)) & (k < 16)
                hi = (iota == (k % 16)) & (k >= 16)
                rv0 = jnp.where(lo, m, rv0)
                rv1 = jnp.where(hi, m, rv1)
                ri0 = jnp.where(lo, gi, ri0)
                ri1 = jnp.where(hi, gi, ri1)
                return (rv0, rv1, ri0, ri1)

            _, _, ri0, ri1 = lax.fori_loop(
                0, _TOP_K, merge_step, (z16f, z16f, z16i, z16i))
            outb[pl.ds(0, 16)] = ri0
            outb[pl.ds(16, 16)] = ri1
            pltpu.sync_copy(outb, out_hbm)


def _topk_body(mass_ref, idx_ref):
    m = mass_ref[...]
    iota = jax.lax.broadcasted_iota(jnp.int32, (1, _M), 1)
    lanes = jax.lax.broadcasted_iota(jnp.int32, (1, _TOP_K), 1)

    def step(i, carry):
        m, inds = carry
        cm = jnp.max(m)
        idx = jnp.min(jnp.where(m == cm, iota, jnp.int32(2 ** 30)))
        inds = jnp.where(lanes == i, idx, inds)
        m = jnp.where(iota == idx, -jnp.inf, m)
        return (m, inds)

    _, inds = jax.lax.fori_loop(
        0, _TOP_K, step, (m, jnp.zeros((1, _TOP_K), jnp.int32)))
    idx_ref[...] = inds


def kernel(hidden, beliefs, goal_embeddings, goal_priorities, norm_weight,
           depth_bias, W_q, W_out, W_gate, b_gate, W_util, W_obs, W_write,
           W_conf):
    B, T, H, M, D, G, NH = _B, _T, _H, _M, _D, _G, _NH
    wcat_bf, bel_bf, belT_bf, bias_row = pl.pallas_call(
        _prep_body,
        out_shape=(
            jax.ShapeDtypeStruct((H, 3 * H + 2 * D + 2), _bf),
            jax.ShapeDtypeStruct((M, D), _bf),
            jax.ShapeDtypeStruct((D, M), _bf),
            jax.ShapeDtypeStruct((1, M), _f32),
        ),
    )(W_q, W_out, W_util, W_obs, W_write, W_gate, W_conf, beliefs,
      goal_embeddings, goal_priorities.reshape(1, G),
      depth_bias.reshape(1, 1))
    nw = norm_weight.reshape(1, H)
    bg = b_gate.reshape(1, 1)

    grid = T // _TT

    out_shapes = (
        jax.ShapeDtypeStruct((B, T, H), _f32),
        jax.ShapeDtypeStruct((B, T, D), _f32),
        jax.ShapeDtypeStruct((B, T, 1), _f32),
        jax.ShapeDtypeStruct((B, T, H), _f32),
        jax.ShapeDtypeStruct((B, T, NH, M), _f32),
        jax.ShapeDtypeStruct((B, T, NH, D), _f32),
        jax.ShapeDtypeStruct((B, T, D), _f32),
        jax.ShapeDtypeStruct((1, M), _f32),
    )
    full = lambda shape: pl.BlockSpec(shape, lambda t: (0,) * len(shape))
    row = lambda last: pl.BlockSpec((1, _TT, last), lambda t: (0, t, 0))
    row4 = lambda last: pl.BlockSpec((1, _TT, _NH, last),
                                     lambda t: (0, t, 0, 0))

    outs = pl.pallas_call(
        _main_body,
        grid=(grid,),
        in_specs=[
            row(H),
            full((D, M)),
            full((M, D)),
            full((1, M)),
            full((1, H)),
            full((H, 3 * H + 2 * D + 2)),
            full((1, 1)),
        ],
        out_specs=[
            row(H), row(D), row(1), row(H),
            row4(M), row4(D), row(D),
            full((1, M)),
        ],
        out_shape=out_shapes,
        scratch_shapes=[pltpu.VMEM((_R, _M), _f32)],
    )(hidden, belT_bf, bel_bf, bias_row, nw, wcat_bf, bg)

    (hidden_out, write_vec, confidence, utility_logits, attn_weights,
     retrieved, obs_vectors, mass) = outs

    topk_call = pl.kernel(
        _sc_topk,
        out_type=(
            jax.ShapeDtypeStruct((_TOP_K,), jnp.int32),
            jax.ShapeDtypeStruct((_NSUB * _TOP_K,), _f32),
            jax.ShapeDtypeStruct((_NSUB * _TOP_K,), jnp.int32),
        ),
        scratch_types=[
            pltpu.VMEM((_SEG,), _f32),
            pltpu.VMEM((_TOP_K,), _f32),
            pltpu.VMEM((_TOP_K,), jnp.int32),
            pltpu.VMEM((_NSUB * _TOP_K,), _f32),
            pltpu.VMEM((_NSUB * _TOP_K,), jnp.int32),
            pltpu.VMEM((_TOP_K,), jnp.int32),
        ],
        mesh=plsc.VectorSubcoreMesh(core_axis_name="c", subcore_axis_name="s"),
        compiler_params=pltpu.CompilerParams(needs_layout_passes=False),
    )
    read_indices, _, _ = topk_call(mass.reshape(M))

    return (hidden_out, write_vec, confidence, utility_logits, read_indices,
            attn_weights, retrieved, obs_vectors)


# speedup vs baseline: 1.0791x; 1.0031x over previous
import jax
import jax.numpy as jnp
from jax import lax
from jax.experimental import pallas as pl
from jax.experimental.pallas import tpu as pltpu
from jax.experimental.pallas import tpu_sc as plsc

_B, _T, _H = 1, 2048, 1024
_M, _D = 4096, 256
_G = 16
_NH = 4
_TOP_K = 32

_TT = 128
_R = _TT * _NH
_CH = 1024
_NCH = _M // _CH

_bf = jnp.bfloat16
_f32 = jnp.float32


def _dot(a, b):
    return jax.lax.dot_general(
        a.astype(_bf), b if b.dtype == _bf else b.astype(_bf),
        (((1,), (0,)), ((), ())), preferred_element_type=_f32)


def _main_body(hid_ref, belT_ref, bel_ref, bias_ref, nw_ref, wcat_ref,
               bg_ref, ho_ref, wv_ref, conf_ref, util_ref, attn_ref,
               retr_ref, obs_ref, mass_ref, scr_ref):
    t = pl.program_id(0)

    @pl.when(t == 0)
    def _init():
        mass_ref[...] = jnp.zeros_like(mass_ref)

    x = hid_ref[0]
    v = jnp.mean(x * x, axis=-1, keepdims=True)
    normed = x * jax.lax.rsqrt(v + 1e-6) * nw_ref[...]

    q = _dot(normed, wcat_ref[:, 0:_H])
    q4 = (q * 0.0625).reshape(_R, _D).astype(_bf)

    mcs, sums = [], []
    for c in range(_NCH):
        sl = slice(c * _CH, (c + 1) * _CH)
        raw = jax.lax.dot_general(
            q4, belT_ref[:, sl],
            (((1,), (0,)), ((), ())), preferred_element_type=_f32)
        s = raw + bias_ref[:, sl]
        m_c = jnp.max(s, axis=-1, keepdims=True)
        e = jnp.exp(s - m_c)
        sums.append(jnp.sum(e, axis=-1, keepdims=True))
        mcs.append(m_c)
        scr_ref[:, sl] = e
    m = mcs[0]
    for c in range(1, _NCH):
        m = jnp.maximum(m, mcs[c])
    corr = [jnp.exp(mc - m) for mc in mcs]
    total = corr[0] * sums[0]
    for c in range(1, _NCH):
        total = total + corr[c] * sums[c]
    sinv = 1.0 / total
    racc = jnp.zeros((_R, _D), _f32)
    for c in range(_NCH):
        sl = slice(c * _CH, (c + 1) * _CH)
        a = scr_ref[:, sl] * (corr[c] * sinv)
        attn_ref[0, :, :, sl] = a.reshape(_TT, _NH, _CH)
        mass_ref[:, sl] += jnp.sum(a, axis=0, keepdims=True)
        racc = racc + _dot(a, bel_ref[sl, :])
    retr_ref[0] = racc.reshape(_TT, _NH, _D)

    rflat = racc.reshape(_TT, _NH * _D)
    binfo = _dot(rflat, wcat_ref[:, _H:2 * _H])
    gc = _dot(normed, wcat_ref[:, 3 * _H + 2 * _D:3 * _H + 2 * _D + 2])
    gate = jax.nn.sigmoid(gc[:, 0:1] + bg_ref[0, 0])
    conf_ref[0] = jax.nn.sigmoid(gc[:, 1:2])
    ho_ref[0] = x + binfo * gate
    util_ref[0] = _dot(normed, wcat_ref[:, 2 * _H:3 * _H])
    obs_ref[0] = _dot(normed, wcat_ref[:, 3 * _H:3 * _H + _D])
    wv_ref[0] = _dot(normed, wcat_ref[:, 3 * _H + _D:3 * _H + 2 * _D])


_NSUB = 16
_SEG = _M // _NSUB
_NV = _SEG // 16
_NEG = -3.0e38
_BIG = 2 ** 30


def _prep_body(wq, wout, wutil, wobs, wwrite, wgate, wconf, bel, ge, prio,
               db, wcat_o, bel_o, belT_o, bias_o):
    wcat_o[:, 0:_H] = wq[...].astype(_bf)
    wcat_o[:, _H:2 * _H] = wout[...].astype(_bf)
    wcat_o[:, 2 * _H:3 * _H] = wutil[...].astype(_bf)
    wcat_o[:, 3 * _H:3 * _H + _D] = wobs[...].astype(_bf)
    wcat_o[:, 3 * _H + _D:3 * _H + 2 * _D] = wwrite[...].astype(_bf)
    wcat_o[:, 3 * _H + 2 * _D:3 * _H + 2 * _D + 1] = wgate[...].astype(_bf)
    wcat_o[:, 3 * _H + 2 * _D + 1:3 * _H + 2 * _D + 2] = wconf[...].astype(_bf)
    b = bel[...].astype(_bf)
    bel_o[...] = b
    bT = b.T
    belT_o[...] = bT
    gb1 = jax.lax.dot_general(
        b, ge[...].astype(_bf),
        (((1,), (1,)), ((), ())), preferred_element_type=_f32)
    gb = jax.lax.dot_general(
        prio[...].astype(_bf), gb1.astype(_bf),
        (((1,), (1,)), ((), ())), preferred_element_type=_f32)
    bias_o[...] = gb * (1.0 / _G) + db[0, 0]


def _vmaxall(x):
    s = plsc.cummax(x)
    return plsc.cummax(lax.rev(s, (0,)))


def _vminall_i32(x):
    return -_vmaxall(-x)


def _sc_topk(mass_hbm, out_hbm, stagev_hbm, stagei_hbm,
             xv, bufv, bufi, mv, mi, outb):
    cid = lax.axis_index("c")
    sid = lax.axis_index("s")
    iota = lax.iota(jnp.int32, 16)

    @pl.when(cid == 0)
    def _():
        base = sid * _SEG
        pltpu.sync_copy(mass_hbm.at[pl.ds(base, _SEG)], xv)

        def local_step(k, res):
            rv0, rv1, ri0, ri1 = res
            vm = xv[pl.ds(0, 16)]
            for v in range(1, _NV):
                vm = jnp.maximum(vm, xv[pl.ds(v * 16, 16)])
            m = _vmaxall(vm)
            cand = jnp.full((16,), _BIG, jnp.int32)
            for v in range(_NV):
                xvv = xv[pl.ds(v * 16, 16)]
                cand = jnp.minimum(
                    cand, jnp.where(xvv == m, iota + (base + v * 16), _BIG))
            gi = _vminall_i32(cand)
            for v in range(_NV):
                xvv = xv[pl.ds(v * 16, 16)]
                xv[pl.ds(v * 16, 16)] = jnp.where(
                    iota + (base + v * 16) == gi, _NEG, xvv)
            lo = (iota == (k % 16)) & (k < 16)
            hi = (iota == (k % 16)) & (k >= 16)
            rv0 = jnp.where(lo, m, rv0)
            rv1 = jnp.where(hi, m, rv1)
            ri0 = jnp.where(lo, gi, ri0)
            ri1 = jnp.where(hi, gi, ri1)
            return (rv0, rv1, ri0, ri1)

        z16f = jnp.full((16,), _NEG, _f32)
        z16i = jnp.zeros((16,), jnp.int32)
        rv0, rv1, ri0, ri1 = lax.fori_loop(
            0, _TOP_K, local_step, (z16f, z16f, z16i, z16i))
        bufv[pl.ds(0, 16)] = rv0
        bufv[pl.ds(16, 16)] = rv1
        bufi[pl.ds(0, 16)] = ri0
        bufi[pl.ds(16, 16)] = ri1
        pltpu.sync_copy(bufv, stagev_hbm.at[pl.ds(sid * _TOP_K, _TOP_K)])
        pltpu.sync_copy(bufi, stagei_hbm.at[pl.ds(sid * _TOP_K, _TOP_K)])
        plsc.subcore_barrier()

        @pl.when(sid == 0)
        def _merge():
            pltpu.sync_copy(stagev_hbm, mv)
            pltpu.sync_copy(stagei_hbm, mi)
            npool = _NSUB * _TOP_K // 16

            def merge_step(k, res):
                rv0, rv1, ri0, ri1 = res
                vm = mv[pl.ds(0, 16)]
                for v in range(1, npool):
                    vm = jnp.maximum(vm, mv[pl.ds(v * 16, 16)])
                m = _vmaxall(vm)
                cand = jnp.full((16,), _BIG, jnp.int32)
                for v in range(npool):
                    vv = mv[pl.ds(v * 16, 16)]
                    iv = mi[pl.ds(v * 16, 16)]
                    cand = jnp.minimum(cand, jnp.where(vv == m, iv, _BIG))
                gi = _vminall_i32(cand)
                for v in range(npool):
                    vv = mv[pl.ds(v * 16, 16)]
                    iv = mi[pl.ds(v * 16, 16)]
                    mv[pl.ds(v * 16, 16)] = jnp.where(
                        (vv == m) & (iv == gi), _NEG, vv)
                lo = (iota == (k % 16)) & (k < 16)
                hi = (iota == (k % 16)) & (k >= 16)
                rv0 = jnp.where(lo, m, rv0)
                rv1 = jnp.where(hi, m, rv1)
                ri0 = jnp.where(lo, gi, ri0)
                ri1 = jnp.where(hi, gi, ri1)
                return (rv0, rv1, ri0, ri1)

            _, _, ri0, ri1 = lax.fori_loop(
                0, _TOP_K, merge_step, (z16f, z16f, z16i, z16i))
            outb[pl.ds(0, 16)] = ri0
            outb[pl.ds(16, 16)] = ri1
            pltpu.sync_copy(outb, out_hbm)


def kernel(hidden, beliefs, goal_embeddings, goal_priorities, norm_weight,
           depth_bias, W_q, W_out, W_gate, b_gate, W_util, W_obs, W_write,
           W_conf):
    B, T, H, M, D, G, NH = _B, _T, _H, _M, _D, _G, _NH
    wcat_bf, bel_bf, belT_bf, bias_row = pl.pallas_call(
        _prep_body,
        out_shape=(
            jax.ShapeDtypeStruct((H, 3 * H + 2 * D + 2), _bf),
            jax.ShapeDtypeStruct((M, D), _bf),
            jax.ShapeDtypeStruct((D, M), _bf),
            jax.ShapeDtypeStruct((1, M), _f32),
        ),
    )(W_q, W_out, W_util, W_obs, W_write, W_gate, W_conf, beliefs,
      goal_embeddings, goal_priorities.reshape(1, G),
      depth_bias.reshape(1, 1))
    nw = norm_weight.reshape(1, H)
    bg = b_gate.reshape(1, 1)

    grid = T // _TT

    out_shapes = (
        jax.ShapeDtypeStruct((B, T, H), _f32),
        jax.ShapeDtypeStruct((B, T, D), _f32),
        jax.ShapeDtypeStruct((B, T, 1), _f32),
        jax.ShapeDtypeStruct((B, T, H), _f32),
        jax.ShapeDtypeStruct((B, T, NH, M), _f32),
        jax.ShapeDtypeStruct((B, T, NH, D), _f32),
        jax.ShapeDtypeStruct((B, T, D), _f32),
        jax.ShapeDtypeStruct((1, M), _f32),
    )
    full = lambda shape: pl.BlockSpec(shape, lambda t: (0,) * len(shape))
    row = lambda last: pl.BlockSpec((1, _TT, last), lambda t: (0, t, 0))
    row4 = lambda last: pl.BlockSpec((1, _TT, _NH, last),
                                     lambda t: (0, t, 0, 0))

    outs = pl.pallas_call(
        _main_body,
        grid=(grid,),
        in_specs=[
            row(H),
            full((D, M)),
            full((M, D)),
            full((1, M)),
            full((1, H)),
            full((H, 3 * H + 2 * D + 2)),
            full((1, 1)),
        ],
        out_specs=[
            row(H), row(D), row(1), row(H),
            row4(M), row4(D), row(D),
            full((1, M)),
        ],
        out_shape=out_shapes,
        scratch_shapes=[pltpu.VMEM((_R, _M), _f32)],
    )(hidden, belT_bf, bel_bf, bias_row, nw, wcat_bf, bg)

    (hidden_out, write_vec, confidence, utility_logits, attn_weights,
     retrieved, obs_vectors, mass) = outs

    topk_call = pl.kernel(
        _sc_topk,
        out_type=(
            jax.ShapeDtypeStruct((_TOP_K,), jnp.int32),
            jax.ShapeDtypeStruct((_NSUB * _TOP_K,), _f32),
            jax.ShapeDtypeStruct((_NSUB * _TOP_K,), jnp.int32),
        ),
        scratch_types=[
            pltpu.VMEM((_SEG,), _f32),
            pltpu.VMEM((_TOP_K,), _f32),
            pltpu.VMEM((_TOP_K,), jnp.int32),
            pltpu.VMEM((_NSUB * _TOP_K,), _f32),
            pltpu.VMEM((_NSUB * _TOP_K,), jnp.int32),
            pltpu.VMEM((_TOP_K,), jnp.int32),
        ],
        mesh=plsc.VectorSubcoreMesh(core_axis_name="c", subcore_axis_name="s"),
        compiler_params=pltpu.CompilerParams(needs_layout_passes=False),
    )
    read_indices, _, _ = topk_call(mass.reshape(M))

    return (hidden_out, write_vec, confidence, utility_logits, read_indices,
            attn_weights, retrieved, obs_vectors)
